# Initial kernel scaffold; baseline (speedup 1.0000x reference)
import jax
import jax.numpy as jnp
from jax.experimental import pallas as pl


def kernel(x, b1_w1, b1_scale1, b1_bias1, b1_w2, b1_scale2, b1_bias2, b2_w1, b2_scale1, b2_bias1, b2_w2, b2_scale2, b2_bias2):
    raise NotImplementedError("write your pallas kernel here")


# speedup vs baseline: 42.8235x; 42.8235x over previous
import functools

import jax
import jax.numpy as jnp
from jax.experimental import pallas as pl
from jax.experimental.pallas import tpu as pltpu


def _body(x_ref, wb_ref, b_ref, o_ref, *, H):
    M, WC = x_ref.shape

    row = jax.lax.broadcasted_iota(jnp.int32, (M, WC), 0) % H
    m_prev = (row > 0).astype(jnp.bfloat16)
    m_next = (row < (H - 1)).astype(jnp.bfloat16)

    def conv_bn(a_bf, i):
        p = pltpu.roll(a_bf, 1, axis=0) * m_prev
        n = pltpu.roll(a_bf, M - 1, axis=0) * m_next
        y = jnp.dot(p, wb_ref[3 * i], preferred_element_type=jnp.float32)
        y += jnp.dot(a_bf, wb_ref[3 * i + 1], preferred_element_type=jnp.float32)
        y += jnp.dot(n, wb_ref[3 * i + 2], preferred_element_type=jnp.float32)
        return y + b_ref[i]

    x0 = x_ref[...]
    h1 = jnp.maximum(conv_bn(x0.astype(jnp.bfloat16), 0), 0.0)
    x1 = jnp.maximum(x0 + conv_bn(h1.astype(jnp.bfloat16), 1), 0.0)
    h2 = jnp.maximum(conv_bn(x1.astype(jnp.bfloat16), 2), 0.0)
    x2 = jnp.maximum(x1 + conv_bn(h2.astype(jnp.bfloat16), 3), 0.0)
    o_ref[...] = x2


def _band(w, scale, eyes, W):
    c = w.shape[-1]
    ws = w * scale[None, None, None, :]
    b = jnp.einsum("xab,yxcd->yacbd", eyes, ws)
    return b.reshape(3, W * c, W * c)


def kernel(x, b1_w1, b1_scale1, b1_bias1, b1_w2, b1_scale2, b1_bias2,
           b2_w1, b2_scale1, b2_bias1, b2_w2, b2_scale2, b2_bias2):
    N, H, W, C = x.shape
    WC = W * C
    B = 32 if N % 32 == 0 else N
    M = B * H

    eyes = jnp.stack([jnp.eye(W, k=1 - kx, dtype=jnp.float32)
                      for kx in range(3)])
    wb = jnp.concatenate([
        _band(b1_w1, b1_scale1, eyes, W),
        _band(b1_w2, b1_scale2, eyes, W),
        _band(b2_w1, b2_scale1, eyes, W),
        _band(b2_w2, b2_scale2, eyes, W),
    ], axis=0).astype(jnp.bfloat16)
    bias = jnp.stack([
        jnp.tile(b1_bias1, W), jnp.tile(b1_bias2, W),
        jnp.tile(b2_bias1, W), jnp.tile(b2_bias2, W),
    ]).reshape(4, 1, WC).astype(jnp.float32)

    x2d = x.reshape(N * H, WC)
    out = pl.pallas_call(
        functools.partial(_body, H=H),
        out_shape=jax.ShapeDtypeStruct((N * H, WC), jnp.float32),
        grid=(N // B,),
        in_specs=[
            pl.BlockSpec((M, WC), lambda n: (n, 0)),
            pl.BlockSpec((12, WC, WC), lambda n: (0, 0, 0)),
            pl.BlockSpec((4, 1, WC), lambda n: (0, 0, 0)),
        ],
        out_specs=pl.BlockSpec((M, WC), lambda n: (n, 0)),
        compiler_params=pltpu.CompilerParams(
            dimension_semantics=("parallel",),
            vmem_limit_bytes=48 * 1024 * 1024,
        ),
    )(x2d, wb, bias)
    return out.reshape(N, H, W, C)
